# Initial kernel scaffold; baseline (speedup 1.0000x reference)
import jax
import jax.numpy as jnp
from jax.experimental import pallas as pl


def kernel(x, table):
    raise NotImplementedError("write your pallas kernel here")


# speedup vs baseline: 1.2883x; 1.2883x over previous
import functools

import jax
import jax.numpy as jnp
from jax import lax
from jax.experimental import pallas as pl
from jax.experimental.pallas import tpu as pltpu
from jax.experimental.pallas import tpu_sc as plsc

NUM_CORES = 2
NUM_SUBCORES = 16
NW = NUM_CORES * NUM_SUBCORES

EMBED_DIM = 256
B_TOTAL = 4096 * 50
CHUNK = 128
PER_W = B_TOTAL // NW
G = PER_W // CHUNK
NBUF = 2


def _sc_gather(x3d, table):
    mesh = plsc.VectorSubcoreMesh(core_axis_name="c", subcore_axis_name="s")

    @functools.partial(
        pl.kernel,
        mesh=mesh,
        out_type=jax.ShapeDtypeStruct((B_TOTAL, EMBED_DIM), jnp.float32),
        scratch_types=[
            pltpu.VMEM((G, CHUNK), jnp.int32),
            pltpu.VMEM((CHUNK, EMBED_DIM), jnp.float32),
            pltpu.VMEM((CHUNK, EMBED_DIM), jnp.float32),
            pltpu.SemaphoreType.DMA,
            pltpu.SemaphoreType.DMA,
        ],
    )
    def k(x_hbm, table_hbm, out_hbm, idx_v, rows0, rows1, gsem, osem):
        wid = lax.axis_index("s") * NUM_CORES + lax.axis_index("c")
        out_base = wid * PER_W
        bufs = (rows0, rows1)

        pltpu.sync_copy(x_hbm.at[wid], idx_v)

        def gather_start(g, buf):
            pltpu.async_copy(table_hbm.at[idx_v.at[g]], buf, gsem)

        def gather_wait(buf):
            pltpu.make_async_copy(table_hbm.at[idx_v.at[0]], buf, gsem).wait()

        def write_start(g, buf):
            pltpu.async_copy(buf, out_hbm.at[pl.ds(out_base + g * CHUNK, CHUNK)], osem)

        def write_wait(buf):
            pltpu.make_async_copy(buf, out_hbm.at[pl.ds(out_base, CHUNK)], osem).wait()

        gather_start(0, bufs[0])
        gather_start(1, bufs[1])

        def body(go, carry):
            for b in range(NBUF):
                g = go * NBUF + b
                buf = bufs[b]
                gather_wait(buf)
                write_start(g, buf)
                write_wait(buf)
                gather_start(g + NBUF, buf)
            return carry

        lax.fori_loop(0, (G - NBUF) // NBUF, body, 0)

        for b in range(NBUF):
            g = G - NBUF + b
            buf = bufs[b]
            gather_wait(buf)
            write_start(g, buf)
            write_wait(buf)

    return k(x3d, table)


def kernel(x, table):
    n, s = x.shape
    x3d = x.reshape(NW, G, CHUNK).astype(jnp.int32)
    out = _sc_gather(x3d, table)
    return out.reshape(n, s, EMBED_DIM)
